# baseline (speedup 1.0000x reference)
import functools

import jax
import jax.numpy as jnp
from jax import lax
from jax.experimental import pallas as pl
from jax.experimental.pallas import tpu as pltpu
from jax.experimental.pallas import tpu_sc as plsc

_SC_PARAMS = pltpu.CompilerParams(
    needs_layout_passes=False, use_tc_tiling_on_sc=False)

N = 10000
E = 160000
D = 256
H = 8
F = 64
HF = H * F
NEG_SLOPE = 0.2

NC = 2
NS = 16
NT = NC * NS
L = 16

NPAD = 10240
EPAD = 163840
EPT = EPAD // NT
CE2 = 16384
NCHUNK = EPAD // CE2
SDB = 14
CT = 4
RROUNDS = HF // (CT * NT)
HR = 2
BN = 512


def _mm_body(feat_ref, w_ref, al_ref, ar_ref, pT_ref, elT_ref, erT_ref):
    p = lax.dot_general(
        w_ref[...], feat_ref[...], (((0,), (1,)), ((), ())),
        preferred_element_type=jnp.float32,
        precision=lax.Precision.HIGHEST,
    )
    b3 = lax.bitcast_convert_type(
        p.astype(jnp.bfloat16), jnp.uint16).astype(jnp.uint32)
    b3 = b3.reshape(HF // 2, 2, BN)
    packed = (b3[:, 0, :] << 16) | b3[:, 1, :]
    pT_ref[...] = lax.bitcast_convert_type(packed, jnp.int32)
    p3 = p.reshape(H, F, BN)
    elT_ref[...] = (p3 * al_ref[...].reshape(H, F, 1)).sum(axis=1)
    erT_ref[...] = (p3 * ar_ref[...].reshape(H, F, 1)).sum(axis=1)


def _project(feat, W, al, ar):
    grid = (NPAD // BN,)
    return pl.pallas_call(
        _mm_body,
        grid=grid,
        in_specs=[
            pl.BlockSpec((BN, D), lambda i: (i, 0)),
            pl.BlockSpec((D, HF), lambda i: (0, 0)),
            pl.BlockSpec((H, F), lambda i: (0, 0)),
            pl.BlockSpec((H, F), lambda i: (0, 0)),
        ],
        out_specs=[
            pl.BlockSpec((HF // 2, BN), lambda i: (0, i)),
            pl.BlockSpec((H, BN), lambda i: (0, i)),
            pl.BlockSpec((H, BN), lambda i: (0, i)),
        ],
        out_shape=[
            jax.ShapeDtypeStruct((HF // 2, NPAD), jnp.int32),
            jax.ShapeDtypeStruct((H, NPAD), jnp.float32),
            jax.ShapeDtypeStruct((H, NPAD), jnp.float32),
        ],
    )(feat, W, al, ar)


def _phase1_body(elT_ref, erT_ref, ei_ref, wT_ref, dp_ref, sd_ref,
                 el_v, er_v, dacc, s_v, d_v, wbuf, sdbuf, sems1):
    core = lax.axis_index("c")
    sub = lax.axis_index("s")
    tid = sub * NC + core
    EPR = E // NT
    ebase = tid * EPT
    pltpu.sync_copy(ei_ref.at[0, pl.ds(tid * EPR, EPR)], s_v.at[pl.ds(0, EPR)])
    pltpu.sync_copy(ei_ref.at[1, pl.ds(tid * EPR, EPR)], d_v.at[pl.ds(0, EPR)])

    @plsc.parallel_loop(0, EPT // L, unroll=8)
    def _pack(g):
        eidx = g * L + lax.iota(jnp.int32, L)
        valid = eidx < EPR
        s16 = jnp.where(valid, s_v[pl.ds(g * L, L)], N)
        d16 = jnp.where(valid, d_v[pl.ds(g * L, L)], N)
        s_v[pl.ds(g * L, L)] = s16
        d_v[pl.ds(g * L, L)] = d16
        sdbuf[pl.ds(g * L, L)] = s16 + d16 * (2 ** SDB)
    pltpu.sync_copy(sdbuf, sd_ref.at[pl.ds(ebase, EPT)])

    zeros16 = jnp.zeros((L,), jnp.float32)

    def start_round(hr, slot):
        hc = min(hr, H // HR - 1)
        pltpu.async_copy(
            elT_ref.at[pl.ds(hc * HR * NPAD, HR * NPAD)],
            el_v.at[pl.ds(slot * HR * NPAD, HR * NPAD)], sems1.at[slot, 0])
        pltpu.async_copy(
            erT_ref.at[pl.ds(hc * HR * NPAD, HR * NPAD)],
            er_v.at[pl.ds(slot * HR * NPAD, HR * NPAD)], sems1.at[slot, 1])

    def drain_round(slot):
        pltpu.make_async_copy(
            elT_ref.at[pl.ds(0, HR * NPAD)],
            el_v.at[pl.ds(slot * HR * NPAD, HR * NPAD)],
            sems1.at[slot, 0]).wait()
        pltpu.make_async_copy(
            erT_ref.at[pl.ds(0, HR * NPAD)],
            er_v.at[pl.ds(slot * HR * NPAD, HR * NPAD)],
            sems1.at[slot, 1]).wait()

    start_round(0, 0)
    for hr in range(H // HR):
        slot = hr % 2
        drain_round(slot)
        start_round(hr + 1, 1 - slot)
        sbase = slot * HR * NPAD

        @plsc.parallel_loop(0, HR * NPAD // L, unroll=8)
        def _zero(i):
            dacc[pl.ds(i * L, L)] = zeros16

        @plsc.parallel_loop(0, EPT // L, unroll=4)
        def _grp(g):
            s16 = s_v[pl.ds(g * L, L)]
            d16 = d_v[pl.ds(g * L, L)]
            for c in range(HR):
                ev = plsc.load_gather(el_v, [s16 + (sbase + c * NPAD)])
                rv = plsc.load_gather(er_v, [d16 + (sbase + c * NPAD)])
                e = ev + rv
                e = jnp.where(e >= 0.0, e, e * NEG_SLOPE)
                w = jnp.exp(e)
                wbuf[pl.ds(c * EPT + g * L, L)] = w
                plsc.addupdate_scatter(dacc, [d16 + (c * NPAD)], w)

        for c in range(HR):
            pltpu.sync_copy(
                wbuf.at[pl.ds(c * EPT, EPT)],
                wT_ref.at[pl.ds((hr * HR + c) * EPAD + ebase, EPT)])
        pltpu.sync_copy(
            dacc, dp_ref.at[tid, pl.ds(hr * HR * NPAD, HR * NPAD)])
    drain_round(H // HR % 2)


def _phase1(elT_flat, erT_flat, edge_index):
    mesh = plsc.VectorSubcoreMesh(
        core_axis_name="c", subcore_axis_name="s", num_cores=NC, num_subcores=NS)
    run = pl.kernel(
        _phase1_body,
        out_type=[
            jax.ShapeDtypeStruct((H * EPAD,), jnp.float32),
            jax.ShapeDtypeStruct((NT, H * NPAD), jnp.float32),
            jax.ShapeDtypeStruct((EPAD,), jnp.int32),
        ],
        mesh=mesh,
        compiler_params=_SC_PARAMS,
        scratch_types=[
            pltpu.VMEM((2 * HR * NPAD,), jnp.float32),
            pltpu.VMEM((2 * HR * NPAD,), jnp.float32),
            pltpu.VMEM((HR * NPAD,), jnp.float32),
            pltpu.VMEM((EPT,), jnp.int32),
            pltpu.VMEM((EPT,), jnp.int32),
            pltpu.VMEM((HR * EPT,), jnp.float32),
            pltpu.VMEM((EPT,), jnp.int32),
            pltpu.SemaphoreType.DMA((2, 2)),
        ],
    )
    return run(elT_flat, erT_flat, edge_index)


def _phase2_body(pT_ref, wT_ref, sd_ref, rstT_ref,
                 tab, acc, sd_v, w_v, sems):
    core = lax.axis_index("c")
    sub = lax.axis_index("s")
    tid = sub * NC + core
    zeros16 = jnp.zeros((L,), jnp.float32)

    def start(k, head, slot):
        kc = jnp.minimum(k, NCHUNK - 1)
        pltpu.async_copy(
            sd_ref.at[pl.ds(kc * CE2, CE2)], sd_v.at[slot], sems.at[slot, 0])
        pltpu.async_copy(
            wT_ref.at[pl.ds(head * EPAD + kc * CE2, CE2)], w_v.at[slot],
            sems.at[slot, 1])

    def drain(slot):
        pltpu.make_async_copy(
            sd_ref.at[pl.ds(0, CE2)], sd_v.at[slot], sems.at[slot, 0]).wait()
        pltpu.make_async_copy(
            wT_ref.at[pl.ds(0, CE2)], w_v.at[slot], sems.at[slot, 1]).wait()

    def compute(slot):
        sd_s, w_s = sd_v.at[slot], w_v.at[slot]

        himask = jnp.full((L,), -65536, jnp.int32)

        @plsc.parallel_loop(0, CE2 // L, unroll=16)
        def _grp(g):
            sd16 = sd_s[pl.ds(g * L, L)]
            s16 = lax.bitwise_and(sd16, 2 ** SDB - 1)
            d16 = lax.shift_right_logical(sd16, SDB)
            wv = w_s[pl.ds(g * L, L)]
            for q in range(CT // 2):
                w32 = plsc.load_gather(tab, [s16 + (q * NPAD)])
                v_hi = plsc.bitcast(lax.bitwise_and(w32, himask), jnp.float32)
                v_lo = plsc.bitcast(w32 << 16, jnp.float32)
                plsc.addupdate_scatter(
                    acc, [d16 + (2 * q * NPAD)], v_hi * wv)
                plsc.addupdate_scatter(
                    acc, [d16 + ((2 * q + 1) * NPAD)], v_lo * wv)

    for r in range(RROUNDS):
        b = r * NT + tid
        cb = b * CT
        head = (b * CT) // F
        pltpu.sync_copy(
            pT_ref.at[pl.ds((cb // 2) * NPAD, (CT // 2) * NPAD)], tab)

        @plsc.parallel_loop(0, CT * NPAD // L, unroll=8)
        def _zero(i):
            acc[pl.ds(i * L, L)] = zeros16

        start(jnp.int32(0), head, 0)

        def pair(p, _):
            k0 = p * 2
            drain(0)
            start(k0 + 1, head, 1)
            compute(0)
            drain(1)
            start(k0 + 2, head, 0)
            compute(1)
            return 0
        lax.fori_loop(0, NCHUNK // 2, pair, 0)
        drain(0)
        pltpu.sync_copy(acc, rstT_ref.at[pl.ds(cb * NPAD, CT * NPAD)])


def _phase2(pT_flat, wT_flat, sdp):
    mesh = plsc.VectorSubcoreMesh(
        core_axis_name="c", subcore_axis_name="s", num_cores=NC, num_subcores=NS)
    run = pl.kernel(
        _phase2_body,
        out_type=jax.ShapeDtypeStruct((HF * NPAD,), jnp.float32),
        mesh=mesh,
        compiler_params=_SC_PARAMS,
        scratch_types=[
            pltpu.VMEM(((CT // 2) * NPAD,), jnp.int32),
            pltpu.VMEM((CT * NPAD,), jnp.float32),
            pltpu.VMEM((2, CE2), jnp.int32),
            pltpu.VMEM((2, CE2), jnp.float32),
            pltpu.SemaphoreType.DMA((2, 2)),
        ],
    )
    return run(pT_flat, wT_flat, sdp)


def _fin_body(rstT_ref, dp_ref, bias_ref, out_ref):
    denom = dp_ref[...].sum(axis=0)
    safe = jnp.where(denom == 0.0, 1.0, denom)
    rr = rstT_ref[...].reshape(H, F, BN) / safe[:, None, :]
    rr = rr + bias_ref[...].reshape(H, F, 1)
    out_ref[...] = rr.reshape(HF, BN).T


def _finalize(rstT, dparts, bias2d):
    grid = (NPAD // BN,)
    return pl.pallas_call(
        _fin_body,
        grid=grid,
        in_specs=[
            pl.BlockSpec((HF, BN), lambda i: (0, i)),
            pl.BlockSpec((NT, H, BN), lambda i: (0, 0, i)),
            pl.BlockSpec((H, F), lambda i: (0, 0)),
        ],
        out_specs=pl.BlockSpec((BN, HF), lambda i: (i, 0)),
        out_shape=jax.ShapeDtypeStruct((NPAD, HF), jnp.float32),
    )(rstT, dparts, bias2d)


@jax.jit
def kernel(feat, edge_index, W, attn_l, attn_r, bias):
    al = attn_l.reshape(H, F)
    ar = attn_r.reshape(H, F)
    pT, elT, erT = _project(feat, W, al, ar)
    wT_flat, dparts, sdp = _phase1(elT.reshape(-1), erT.reshape(-1), edge_index)
    rstT_flat = _phase2(pT.reshape(-1), wT_flat, sdp)
    rstT = rstT_flat.reshape(HF, NPAD)
    out = _finalize(rstT, dparts.reshape(NT, H, NPAD), bias.reshape(H, F))
    return out[:N].reshape(N, H, F)


# speedup vs baseline: 1.0103x; 1.0103x over previous
import functools

import jax
import jax.numpy as jnp
from jax import lax
from jax.experimental import pallas as pl
from jax.experimental.pallas import tpu as pltpu
from jax.experimental.pallas import tpu_sc as plsc

_SC_PARAMS = pltpu.CompilerParams(
    needs_layout_passes=False, use_tc_tiling_on_sc=False)

N = 10000
E = 160000
D = 256
H = 8
F = 64
HF = H * F
NEG_SLOPE = 0.2

NC = 2
NS = 16
NT = NC * NS
L = 16

NPAD = 10240
EPAD = 163840
EPT = EPAD // NT
CE2 = 16384
NCHUNK = EPAD // CE2
SDB = 14
CT = 4
RROUNDS = HF // (CT * NT)
HR = 2
BN = 512


def _mm_body(feat_ref, w_ref, al_ref, ar_ref, pT_ref, elT_ref, erT_ref):
    p = lax.dot_general(
        w_ref[...], feat_ref[...], (((0,), (1,)), ((), ())),
        preferred_element_type=jnp.float32,
        precision=lax.Precision.HIGHEST,
    )
    b3 = lax.bitcast_convert_type(
        p.astype(jnp.bfloat16), jnp.uint16).astype(jnp.uint32)
    b3 = b3.reshape(HF // 2, 2, BN)
    packed = (b3[:, 0, :] << 16) | b3[:, 1, :]
    pT_ref[...] = lax.bitcast_convert_type(packed, jnp.int32)
    p3 = p.reshape(H, F, BN)
    elT_ref[...] = (p3 * al_ref[...].reshape(H, F, 1)).sum(axis=1)
    erT_ref[...] = (p3 * ar_ref[...].reshape(H, F, 1)).sum(axis=1)


def _project(feat, W, al, ar):
    grid = (NPAD // BN,)
    return pl.pallas_call(
        _mm_body,
        grid=grid,
        in_specs=[
            pl.BlockSpec((BN, D), lambda i: (i, 0)),
            pl.BlockSpec((D, HF), lambda i: (0, 0)),
            pl.BlockSpec((H, F), lambda i: (0, 0)),
            pl.BlockSpec((H, F), lambda i: (0, 0)),
        ],
        out_specs=[
            pl.BlockSpec((HF // 2, BN), lambda i: (0, i)),
            pl.BlockSpec((H, BN), lambda i: (0, i)),
            pl.BlockSpec((H, BN), lambda i: (0, i)),
        ],
        out_shape=[
            jax.ShapeDtypeStruct((HF // 2, NPAD), jnp.int32),
            jax.ShapeDtypeStruct((H, NPAD), jnp.float32),
            jax.ShapeDtypeStruct((H, NPAD), jnp.float32),
        ],
    )(feat, W, al, ar)


def _phase1_body(elT_ref, erT_ref, ei_ref, wT_ref, dp_ref, sd_ref,
                 el_v, er_v, dacc, s_v, d_v, wbuf, sdbuf, sems1):
    core = lax.axis_index("c")
    sub = lax.axis_index("s")
    tid = sub * NC + core
    EPR = E // NT
    ebase = tid * EPT
    pltpu.sync_copy(ei_ref.at[0, pl.ds(tid * EPR, EPR)], s_v.at[pl.ds(0, EPR)])
    pltpu.sync_copy(ei_ref.at[1, pl.ds(tid * EPR, EPR)], d_v.at[pl.ds(0, EPR)])

    @plsc.parallel_loop(0, EPT // L, unroll=8)
    def _pack(g):
        eidx = g * L + lax.iota(jnp.int32, L)
        valid = eidx < EPR
        s16 = jnp.where(valid, s_v[pl.ds(g * L, L)], N)
        d16 = jnp.where(valid, d_v[pl.ds(g * L, L)], N)
        s_v[pl.ds(g * L, L)] = s16
        d_v[pl.ds(g * L, L)] = d16
        sdbuf[pl.ds(g * L, L)] = s16 + d16 * (2 ** SDB)
    pltpu.sync_copy(sdbuf, sd_ref.at[pl.ds(ebase, EPT)])

    zeros16 = jnp.zeros((L,), jnp.float32)

    def start_round(hr, slot):
        hc = min(hr, H // HR - 1)
        pltpu.async_copy(
            elT_ref.at[pl.ds(hc * HR * NPAD, HR * NPAD)],
            el_v.at[pl.ds(slot * HR * NPAD, HR * NPAD)], sems1.at[slot, 0])
        pltpu.async_copy(
            erT_ref.at[pl.ds(hc * HR * NPAD, HR * NPAD)],
            er_v.at[pl.ds(slot * HR * NPAD, HR * NPAD)], sems1.at[slot, 1])

    def drain_round(slot):
        pltpu.make_async_copy(
            elT_ref.at[pl.ds(0, HR * NPAD)],
            el_v.at[pl.ds(slot * HR * NPAD, HR * NPAD)],
            sems1.at[slot, 0]).wait()
        pltpu.make_async_copy(
            erT_ref.at[pl.ds(0, HR * NPAD)],
            er_v.at[pl.ds(slot * HR * NPAD, HR * NPAD)],
            sems1.at[slot, 1]).wait()

    start_round(0, 0)
    for hr in range(H // HR):
        slot = hr % 2
        drain_round(slot)
        start_round(hr + 1, 1 - slot)
        sbase = slot * HR * NPAD

        @plsc.parallel_loop(0, HR * NPAD // L, unroll=8)
        def _zero(i):
            dacc[pl.ds(i * L, L)] = zeros16

        @plsc.parallel_loop(0, EPT // L, unroll=4)
        def _grp(g):
            s16 = s_v[pl.ds(g * L, L)]
            d16 = d_v[pl.ds(g * L, L)]
            for c in range(HR):
                ev = plsc.load_gather(el_v, [s16 + (sbase + c * NPAD)])
                rv = plsc.load_gather(er_v, [d16 + (sbase + c * NPAD)])
                e = ev + rv
                e = jnp.where(e >= 0.0, e, e * NEG_SLOPE)
                w = jnp.exp(e)
                wbuf[pl.ds(c * EPT + g * L, L)] = w
                plsc.addupdate_scatter(dacc, [d16 + (c * NPAD)], w)

        for c in range(HR):
            pltpu.sync_copy(
                wbuf.at[pl.ds(c * EPT, EPT)],
                wT_ref.at[pl.ds((hr * HR + c) * EPAD + ebase, EPT)])
        pltpu.sync_copy(
            dacc, dp_ref.at[tid, pl.ds(hr * HR * NPAD, HR * NPAD)])
    drain_round(H // HR % 2)


def _phase1(elT_flat, erT_flat, edge_index):
    mesh = plsc.VectorSubcoreMesh(
        core_axis_name="c", subcore_axis_name="s", num_cores=NC, num_subcores=NS)
    run = pl.kernel(
        _phase1_body,
        out_type=[
            jax.ShapeDtypeStruct((H * EPAD,), jnp.float32),
            jax.ShapeDtypeStruct((NT, H * NPAD), jnp.float32),
            jax.ShapeDtypeStruct((EPAD,), jnp.int32),
        ],
        mesh=mesh,
        compiler_params=_SC_PARAMS,
        scratch_types=[
            pltpu.VMEM((2 * HR * NPAD,), jnp.float32),
            pltpu.VMEM((2 * HR * NPAD,), jnp.float32),
            pltpu.VMEM((HR * NPAD,), jnp.float32),
            pltpu.VMEM((EPT,), jnp.int32),
            pltpu.VMEM((EPT,), jnp.int32),
            pltpu.VMEM((HR * EPT,), jnp.float32),
            pltpu.VMEM((EPT,), jnp.int32),
            pltpu.SemaphoreType.DMA((2, 2)),
        ],
    )
    return run(elT_flat, erT_flat, edge_index)


def _phase2_body(pT_ref, wT_ref, sd_ref, rstT_ref,
                 tab, acc, sd_v, w_v, sems):
    core = lax.axis_index("c")
    sub = lax.axis_index("s")
    tid = sub * NC + core
    zeros16 = jnp.zeros((L,), jnp.float32)

    def start(k, head, slot):
        kc = jnp.minimum(k, NCHUNK - 1)
        pltpu.async_copy(
            sd_ref.at[pl.ds(kc * CE2, CE2)], sd_v.at[slot], sems.at[slot, 0])
        pltpu.async_copy(
            wT_ref.at[pl.ds(head * EPAD + kc * CE2, CE2)], w_v.at[slot],
            sems.at[slot, 1])

    def drain(slot):
        pltpu.make_async_copy(
            sd_ref.at[pl.ds(0, CE2)], sd_v.at[slot], sems.at[slot, 0]).wait()
        pltpu.make_async_copy(
            wT_ref.at[pl.ds(0, CE2)], w_v.at[slot], sems.at[slot, 1]).wait()

    def compute(slot):
        sd_s, w_s = sd_v.at[slot], w_v.at[slot]

        himask = jnp.full((L,), -65536, jnp.int32)

        @plsc.parallel_loop(0, CE2 // L, unroll=8)
        def _grp(g):
            sd16 = sd_s[pl.ds(g * L, L)]
            s16 = lax.bitwise_and(sd16, 2 ** SDB - 1)
            d16 = lax.shift_right_logical(sd16, SDB)
            wv = w_s[pl.ds(g * L, L)]
            for q in range(CT // 2):
                w32 = plsc.load_gather(tab, [s16 + (q * NPAD)])
                v_hi = plsc.bitcast(lax.bitwise_and(w32, himask), jnp.float32)
                v_lo = plsc.bitcast(w32 << 16, jnp.float32)
                plsc.addupdate_scatter(
                    acc, [d16 + (2 * q * NPAD)], v_hi * wv)
                plsc.addupdate_scatter(
                    acc, [d16 + ((2 * q + 1) * NPAD)], v_lo * wv)

    for r in range(RROUNDS):
        b = r * NT + tid
        cb = b * CT
        head = (b * CT) // F
        pltpu.sync_copy(
            pT_ref.at[pl.ds((cb // 2) * NPAD, (CT // 2) * NPAD)], tab)

        @plsc.parallel_loop(0, CT * NPAD // L, unroll=8)
        def _zero(i):
            acc[pl.ds(i * L, L)] = zeros16

        start(jnp.int32(0), head, 0)

        def pair(p, _):
            k0 = p * 2
            drain(0)
            start(k0 + 1, head, 1)
            compute(0)
            drain(1)
            start(k0 + 2, head, 0)
            compute(1)
            return 0
        lax.fori_loop(0, NCHUNK // 2, pair, 0)
        drain(0)
        pltpu.sync_copy(acc, rstT_ref.at[pl.ds(cb * NPAD, CT * NPAD)])


def _phase2(pT_flat, wT_flat, sdp):
    mesh = plsc.VectorSubcoreMesh(
        core_axis_name="c", subcore_axis_name="s", num_cores=NC, num_subcores=NS)
    run = pl.kernel(
        _phase2_body,
        out_type=jax.ShapeDtypeStruct((HF * NPAD,), jnp.float32),
        mesh=mesh,
        compiler_params=_SC_PARAMS,
        scratch_types=[
            pltpu.VMEM(((CT // 2) * NPAD,), jnp.int32),
            pltpu.VMEM((CT * NPAD,), jnp.float32),
            pltpu.VMEM((2, CE2), jnp.int32),
            pltpu.VMEM((2, CE2), jnp.float32),
            pltpu.SemaphoreType.DMA((2, 2)),
        ],
    )
    return run(pT_flat, wT_flat, sdp)


def _fin_body(rstT_ref, dp_ref, bias_ref, out_ref):
    denom = dp_ref[...].sum(axis=0)
    safe = jnp.where(denom == 0.0, 1.0, denom)
    rr = rstT_ref[...].reshape(H, F, BN) / safe[:, None, :]
    rr = rr + bias_ref[...].reshape(H, F, 1)
    out_ref[...] = rr.reshape(HF, BN).T


def _finalize(rstT, dparts, bias2d):
    grid = (NPAD // BN,)
    return pl.pallas_call(
        _fin_body,
        grid=grid,
        in_specs=[
            pl.BlockSpec((HF, BN), lambda i: (0, i)),
            pl.BlockSpec((NT, H, BN), lambda i: (0, 0, i)),
            pl.BlockSpec((H, F), lambda i: (0, 0)),
        ],
        out_specs=pl.BlockSpec((BN, HF), lambda i: (i, 0)),
        out_shape=jax.ShapeDtypeStruct((NPAD, HF), jnp.float32),
    )(rstT, dparts, bias2d)


@jax.jit
def kernel(feat, edge_index, W, attn_l, attn_r, bias):
    al = attn_l.reshape(H, F)
    ar = attn_r.reshape(H, F)
    pT, elT, erT = _project(feat, W, al, ar)
    wT_flat, dparts, sdp = _phase1(elT.reshape(-1), erT.reshape(-1), edge_index)
    rstT_flat = _phase2(pT.reshape(-1), wT_flat, sdp)
    rstT = rstT_flat.reshape(HF, NPAD)
    out = _finalize(rstT, dparts.reshape(NT, H, NPAD), bias.reshape(H, F))
    return out[:N].reshape(N, H, F)
